# Initial kernel scaffold; baseline (speedup 1.0000x reference)
import jax
import jax.numpy as jnp
from jax.experimental import pallas as pl


def kernel(x, edge_index, edge_attr, W_in, b_in, W_l, b_l, W_r, b_r, W_e, att, bias_conv, W_out, b_out):
    raise NotImplementedError("write your pallas kernel here")


# speedup vs baseline: 33.5217x; 33.5217x over previous
import jax
import jax.numpy as jnp
from jax import lax
from jax.experimental import pallas as pl
from jax.experimental.pallas import tpu as pltpu
from jax.experimental.pallas import tpu_sc as plsc

N = 10000
E = 320000
IN_DIM = 128
D = 64
OUT_DIM = 48
NC, NS, L = 2, 16, 16
NW = NC * NS
EPT = E // NW
BLK = 400
NBLK = EPT // BLK
NG = BLK // L
GSUB = BLK // 80
NPAD = 10240
AMX = 2 * NPAD
CHK = AMX // NS
NEG = -1e30

_mesh = plsc.VectorSubcoreMesh(
    core_axis_name="c", subcore_axis_name="s", num_cores=NC, num_subcores=NS
)
_sc_params = pltpu.CompilerParams(
    needs_layout_passes=False, use_tc_tiling_on_sc=False
)


def _proj_body(x_ref, win_ref, bin_ref, wl_ref, bl_ref, wr_ref, br_ref,
               ea2_ref, f_ref, xl_ref, xr_ref, efm_ref):
    h = jnp.dot(x_ref[...], win_ref[...]) + bin_ref[...]
    h = jnp.where(h > 0, h, jnp.exp(h) - 1.0)
    xl_ref[...] = jnp.dot(h, wl_ref[...]) + bl_ref[...]
    xr_ref[...] = jnp.dot(h, wr_ref[...]) + br_ref[...]
    cs = jnp.sum(ea2_ref[...], axis=0, keepdims=True)
    efm_ref[...] = jnp.dot(cs, f_ref[...]) * (1.0 / E)


def _proj(x, W_in, b_in, W_l, b_l, W_r, b_r, ea2, F):
    return pl.pallas_call(
        _proj_body,
        out_shape=[
            jax.ShapeDtypeStruct((N, D), jnp.float32),
            jax.ShapeDtypeStruct((N, D), jnp.float32),
            jax.ShapeDtypeStruct((1, D), jnp.float32),
        ],
        compiler_params=pltpu.CompilerParams(vmem_limit_bytes=100 * 1024 * 1024),
    )(x, W_in, b_in, W_l, b_l, W_r, b_r, ea2, F)


def _sc_alpha_body(src_hbm, dst_hbm, ea_hbm, xl_hbm, xr_hbm, we_hbm, att_hbm,
                   alpha0_hbm, alpha1_hbm, amax_hbm,
                   si, di, ea_b, xl_rows, xr_rows, a0_b, a1_b, amax_priv,
                   loc, we_v, att_v, amax_sh, sem):
    cid = lax.axis_index("c")
    sid = lax.axis_index("s")
    wid = sid * NC + cid
    lane = lax.iota(jnp.int32, L)

    pltpu.sync_copy(we_hbm, we_v)
    pltpu.sync_copy(att_hbm, att_v)

    def init_body(i, _):
        amax_priv[pl.ds(16 * i, 16)] = jnp.full((L,), NEG, jnp.float32)
        return 0

    lax.fori_loop(0, AMX // 16, init_body, 0)

    we = [[we_v[k, pl.ds(16 * v, 16)] for v in range(4)] for k in range(4)]
    at = [[att_v[h, pl.ds(16 * v, 16)] for v in range(2)] for h in range(2)]
    ctake = [jnp.full((L,), k, jnp.int32) for k in range(4)]

    def block_body(b, _):
        base = wid * EPT + b * BLK
        for c in range(GSUB):
            pltpu.sync_copy(src_hbm.at[pl.ds(base + 80 * c, 80)], si.at[c])
            pltpu.sync_copy(dst_hbm.at[pl.ds(base + 80 * c, 80)], di.at[c])
        pltpu.sync_copy(ea_hbm.at[pl.ds(4 * base, 4 * BLK)],
                        ea_b.at[pl.ds(0, 4 * BLK)])
        cps = []
        for c in range(GSUB):
            cps.append(pltpu.async_copy(
                xl_hbm.at[si.at[c]], xl_rows.at[pl.ds(80 * c, 80)], sem))
            cps.append(pltpu.async_copy(
                xr_hbm.at[di.at[c]], xr_rows.at[pl.ds(80 * c, 80)], sem))
        for cp in cps:
            cp.wait()

        def group_body(g, _):
            acc0 = jnp.zeros((L,), jnp.float32)
            acc1 = jnp.zeros((L,), jnp.float32)
            for j in range(L):
                e = L * g + j
                av = ea_b[pl.ds(4 * e, 16)]
                t = [jnp.take(av, ctake[k]) for k in range(4)]
                a0 = jnp.float32(0)
                a1 = jnp.float32(0)
                for v in range(4):
                    ef = (t[0] * we[0][v] + t[1] * we[1][v]
                          + t[2] * we[2][v] + t[3] * we[3][v])
                    m = xl_rows[e, pl.ds(16 * v, 16)] \
                        + xr_rows[e, pl.ds(16 * v, 16)] + ef
                    lr = jnp.where(m >= 0, m, 0.2 * m)
                    if v < 2:
                        a0 = a0 + jnp.sum(lr * at[0][v], axis=0)
                    else:
                        a1 = a1 + jnp.sum(lr * at[1][v - 2], axis=0)
                acc0 = jnp.where(lane == j, a0, acc0)
                acc1 = jnp.where(lane == j, a1, acc1)
            a0_b[pl.ds(L * g, L)] = acc0
            a1_b[pl.ds(L * g, L)] = acc1

            r = g // GSUB
            cc = 16 * (g % GSUB)
            dvec = di[r, pl.ds(cc, 16)]
            for h, acc in ((0, acc0), (1, acc1)):
                kk, vv = plsc.sort_key_val(2 * dvec + h, acc)
                for sh in (1, 2, 4, 8):
                    sl = jnp.maximum(lane - sh, 0)
                    same = (jnp.take(kk, sl) == kk) & (lane >= sh)
                    vv = jnp.where(same, jnp.maximum(vv, jnp.take(vv, sl)), vv)
                nxt = jnp.minimum(lane + 1, L - 1)
                last = (jnp.take(kk, nxt) != kk) | (lane == L - 1)
                cur = plsc.load_gather(amax_priv, [kk])
                plsc.store_scatter(amax_priv, [kk], jnp.maximum(cur, vv),
                                   mask=last)
            return 0

        lax.fori_loop(0, NG, group_body, 0)
        pltpu.sync_copy(a0_b, alpha0_hbm.at[pl.ds(base, BLK)])
        pltpu.sync_copy(a1_b, alpha1_hbm.at[pl.ds(base, BLK)])
        return 0

    lax.fori_loop(0, NBLK, block_body, 0)

    pltpu.sync_copy(amax_priv, amax_sh.at[sid])
    plsc.subcore_barrier()
    pltpu.sync_copy(amax_sh.at[:, pl.ds(CHK * sid, CHK)], loc)

    def comb_body(i, _):
        m = loc[0, pl.ds(16 * i, 16)]
        for r in range(1, NS):
            m = jnp.maximum(m, loc[r, pl.ds(16 * i, 16)])
        ea_b[pl.ds(16 * i, 16)] = m
        return 0

    lax.fori_loop(0, CHK // 16, comb_body, 0)
    pltpu.sync_copy(ea_b.at[pl.ds(0, CHK)],
                    amax_hbm.at[pl.ds(cid * AMX + CHK * sid, CHK)])


def _sc_alpha(src, dst, ea_flat, xl, xr, W_e, att):
    return pl.kernel(
        _sc_alpha_body,
        out_type=[
            jax.ShapeDtypeStruct((E,), jnp.float32),
            jax.ShapeDtypeStruct((E,), jnp.float32),
            jax.ShapeDtypeStruct((NC * AMX,), jnp.float32),
        ],
        mesh=_mesh,
        scratch_types=[
            pltpu.VMEM((GSUB, 80), jnp.int32),
            pltpu.VMEM((GSUB, 80), jnp.int32),
            pltpu.VMEM((4 * BLK + 16,), jnp.float32),
            pltpu.VMEM((BLK, D), jnp.float32),
            pltpu.VMEM((BLK, D), jnp.float32),
            pltpu.VMEM((BLK,), jnp.float32),
            pltpu.VMEM((BLK,), jnp.float32),
            pltpu.VMEM((AMX,), jnp.float32),
            pltpu.VMEM((NS, CHK), jnp.float32),
            pltpu.VMEM((4, D), jnp.float32),
            pltpu.VMEM((2, 32), jnp.float32),
            pltpu.VMEM_SHARED((NS, AMX), jnp.float32),
            pltpu.SemaphoreType.DMA,
        ],
        compiler_params=_sc_params,
    )(src, dst, ea_flat, xl, xr, W_e, att)


def _combine_body(amaxp_ref, xl_ref, xr_ref, efm_ref, att_ref,
                  amax_ref, wself_ref):
    partmax = jnp.max(amaxp_ref[...], axis=0)
    m = xl_ref[...] + xr_ref[...] + efm_ref[...]
    lr = jnp.where(m >= 0, m, 0.2 * m)
    a0 = jnp.sum(lr[:, 0:32] * att_ref[0:1, :], axis=1, keepdims=True)
    a1 = jnp.sum(lr[:, 32:64] * att_ref[1:2, :], axis=1, keepdims=True)
    aself = jnp.concatenate([a0, a1], axis=1)
    aself = jnp.concatenate(
        [aself, jnp.full((NPAD - N, 2), NEG, jnp.float32)], axis=0)
    amax = jnp.maximum(partmax, aself)
    amax_ref[...] = amax
    wself_ref[...] = jnp.exp(aself - amax)


def _combine(amax_part, xl, xr, ef_mean, att):
    return pl.pallas_call(
        _combine_body,
        out_shape=[
            jax.ShapeDtypeStruct((NPAD, 2), jnp.float32),
            jax.ShapeDtypeStruct((NPAD, 2), jnp.float32),
        ],
        compiler_params=pltpu.CompilerParams(vmem_limit_bytes=100 * 1024 * 1024),
    )(amax_part, xl, xr, ef_mean, att)


def _sc_scatter_body(src_hbm, dst_hbm, alpha0_hbm, alpha1_hbm, amax_hbm,
                     xl_hbm, numer_hbm,
                     si, di, xl_rows, msg_rows, a0_b, a1_b, amax_loc,
                     numer_sp, sem):
    cid = lax.axis_index("c")
    sid = lax.axis_index("s")
    wid = sid * NC + cid
    lane = lax.iota(jnp.int32, L)
    cj = [jnp.full((L,), j, jnp.int32) for j in range(L)]

    def zero_body(r, _):
        for v in range(5):
            msg_rows[r, pl.ds(16 * v, 16)] = jnp.zeros((L,), jnp.float32)
        return 0

    lax.fori_loop(0, BLK, zero_body, 0)
    npt = N // NS
    pltpu.sync_copy(msg_rows.at[pl.ds(0, BLK)],
                    numer_sp.at[pl.ds(npt * sid, BLK)])
    pltpu.sync_copy(msg_rows.at[pl.ds(0, npt - BLK)],
                    numer_sp.at[pl.ds(npt * sid + BLK, npt - BLK)])
    pltpu.sync_copy(amax_hbm, amax_loc)
    plsc.subcore_barrier()

    def block_body(b, _):
        base = wid * EPT + b * BLK
        for c in range(GSUB):
            pltpu.sync_copy(src_hbm.at[pl.ds(base + 80 * c, 80)], si.at[c])
            pltpu.sync_copy(dst_hbm.at[pl.ds(base + 80 * c, 80)], di.at[c])
        pltpu.sync_copy(alpha0_hbm.at[pl.ds(base, BLK)], a0_b)
        pltpu.sync_copy(alpha1_hbm.at[pl.ds(base, BLK)], a1_b)
        cps = [pltpu.async_copy(xl_hbm.at[si.at[c]],
                                xl_rows.at[pl.ds(80 * c, 80)], sem)
               for c in range(GSUB)]
        for cp in cps:
            cp.wait()

        def group_body(g, _):
            r = g // GSUB
            cc = 16 * (g % GSUB)
            dvec = di[r, pl.ds(cc, 16)]
            am0 = plsc.load_gather(amax_loc, [2 * dvec])
            am1 = plsc.load_gather(amax_loc, [2 * dvec + 1])
            w0 = jnp.exp(a0_b[pl.ds(L * g, L)] - am0)
            w1 = jnp.exp(a1_b[pl.ds(L * g, L)] - am1)
            for j in range(L):
                e = L * g + j
                w0j = jnp.take(w0, cj[j])
                w1j = jnp.take(w1, cj[j])
                msg_rows[e, pl.ds(0, 16)] = xl_rows[e, pl.ds(0, 16)] * w0j
                msg_rows[e, pl.ds(16, 16)] = xl_rows[e, pl.ds(16, 16)] * w0j
                msg_rows[e, pl.ds(32, 16)] = xl_rows[e, pl.ds(32, 16)] * w1j
                msg_rows[e, pl.ds(48, 16)] = xl_rows[e, pl.ds(48, 16)] * w1j
                den = jnp.where(lane == 0, w0j,
                                jnp.where(lane == 1, w1j, 0.0))
                msg_rows[e, pl.ds(64, 16)] = den
            return 0

        lax.fori_loop(0, NG, group_body, 0)
        for c in range(GSUB):
            pltpu.sync_copy(msg_rows.at[pl.ds(80 * c, 80)],
                            numer_sp.at[di.at[c]], add=True)
        return 0

    lax.fori_loop(0, NBLK, block_body, 0)
    plsc.subcore_barrier()

    @pl.when(sid == 0)
    def _():
        pltpu.sync_copy(numer_sp, numer_hbm.at[pl.ds(cid * N, N)])


def _sc_scatter(src, dst, alpha0, alpha1, amax_flat, xl):
    return pl.kernel(
        _sc_scatter_body,
        out_type=jax.ShapeDtypeStruct((NC * N, 80), jnp.float32),
        mesh=_mesh,
        scratch_types=[
            pltpu.VMEM((GSUB, 80), jnp.int32),
            pltpu.VMEM((GSUB, 80), jnp.int32),
            pltpu.VMEM((BLK, D), jnp.float32),
            pltpu.VMEM((BLK, 80), jnp.float32),
            pltpu.VMEM((BLK,), jnp.float32),
            pltpu.VMEM((BLK,), jnp.float32),
            pltpu.VMEM((AMX,), jnp.float32),
            pltpu.VMEM_SHARED((N, 80), jnp.float32),
            pltpu.SemaphoreType.DMA,
        ],
        compiler_params=_sc_params,
    )(src, dst, alpha0, alpha1, amax_flat, xl)


def _final_body(numer_ref, xl_ref, wself_ref, bias_ref, wout_ref, bout_ref,
                z_ref):
    num = numer_ref[0:N, :] + numer_ref[N:2 * N, :]
    w = wself_ref[0:N, :]
    w0 = w[:, 0:1]
    w1 = w[:, 1:2]
    xlv = xl_ref[...]
    n0 = num[:, 0:32] + w0 * xlv[:, 0:32]
    n1 = num[:, 32:64] + w1 * xlv[:, 32:64]
    d0 = num[:, 64:65] + w0 + 1e-16
    d1 = num[:, 65:66] + w1 + 1e-16
    g = jnp.concatenate([n0 / d0, n1 / d1], axis=1) + bias_ref[...]
    hh = jnp.where(g > 0, g, jnp.exp(g) - 1.0)
    z_ref[...] = jnp.dot(hh, wout_ref[...]) + bout_ref[...]


def _final(numer_part, xl, wself, bias_conv, W_out, b_out):
    return pl.pallas_call(
        _final_body,
        out_shape=jax.ShapeDtypeStruct((N, OUT_DIM), jnp.float32),
        compiler_params=pltpu.CompilerParams(vmem_limit_bytes=100 * 1024 * 1024),
    )(numer_part, xl, wself, bias_conv, W_out, b_out)


def kernel(x, edge_index, edge_attr, W_in, b_in, W_l, b_l, W_r, b_r, W_e,
           att, bias_conv, W_out, b_out):
    src = edge_index[0]
    dst = edge_index[1]
    ea_flat = edge_attr.reshape(-1)
    ea2 = edge_attr.reshape(E // 128, 512)
    F = jnp.tile(W_e, (128, 1))

    xl, xr, ef_mean = _proj(x, W_in, b_in.reshape(1, D), W_l,
                            b_l.reshape(1, D), W_r, b_r.reshape(1, D),
                            ea2, F)
    alpha0, alpha1, amax_sc = _sc_alpha(src, dst, ea_flat, xl, xr, W_e, att)
    amax, wself = _combine(amax_sc.reshape(NC, NPAD, 2), xl, xr,
                           ef_mean, att)
    numer_part = _sc_scatter(src, dst, alpha0, alpha1, amax.reshape(AMX), xl)
    z = _final(numer_part, xl, wself, bias_conv.reshape(1, D), W_out,
               b_out.reshape(1, OUT_DIM))
    return z


# speedup vs baseline: 47.2531x; 1.4096x over previous
import jax
import jax.numpy as jnp
from jax import lax
from jax.experimental import pallas as pl
from jax.experimental.pallas import tpu as pltpu
from jax.experimental.pallas import tpu_sc as plsc

N = 10000
E = 320000
IN_DIM = 128
D = 64
OUT_DIM = 48
NC, NS, L = 2, 16, 16
NW = NC * NS
EPT = E // NW
BLK = 400
NBLK = EPT // BLK
NG = BLK // L
GSUB = BLK // 80
NPAD = 10240
AMX = 2 * NPAD
CHK = AMX // NS
NEG = -1e30

_mesh = plsc.VectorSubcoreMesh(
    core_axis_name="c", subcore_axis_name="s", num_cores=NC, num_subcores=NS
)
_sc_params = pltpu.CompilerParams(
    needs_layout_passes=False, use_tc_tiling_on_sc=False
)


def _proj_body(x_ref, win_ref, bin_ref, wl_ref, bl_ref, wr_ref, br_ref,
               ea2_ref, f_ref, xl_ref, xr_ref, efm_ref):
    h = jnp.dot(x_ref[...], win_ref[...]) + bin_ref[...]
    h = jnp.where(h > 0, h, jnp.exp(h) - 1.0)
    xl_ref[...] = jnp.dot(h, wl_ref[...]) + bl_ref[...]
    xr_ref[...] = jnp.dot(h, wr_ref[...]) + br_ref[...]
    cs = jnp.sum(ea2_ref[...], axis=0, keepdims=True)
    efm_ref[...] = jnp.dot(cs, f_ref[...]) * (1.0 / E)


def _proj(x, W_in, b_in, W_l, b_l, W_r, b_r, ea2, F):
    return pl.pallas_call(
        _proj_body,
        out_shape=[
            jax.ShapeDtypeStruct((N, D), jnp.float32),
            jax.ShapeDtypeStruct((N, D), jnp.float32),
            jax.ShapeDtypeStruct((1, D), jnp.float32),
        ],
        compiler_params=pltpu.CompilerParams(vmem_limit_bytes=100 * 1024 * 1024),
    )(x, W_in, b_in, W_l, b_l, W_r, b_r, ea2, F)


RING = 5
CH = 80
NCH = EPT // CH
CG = CH // L


def _sc_alpha_body(src_hbm, dst_hbm, ea_hbm, xl_hbm, xr_hbm, we_hbm, att_hbm,
                   alpha0_hbm, alpha1_hbm, amax_hbm,
                   si, di, ea_b, xl_rows, xr_rows, a0_b, a1_b, amax_priv,
                   we_v, att_v, amax_sh, sem_i, sem_g, sem_o):
    cid = lax.axis_index("c")
    sid = lax.axis_index("s")
    wid = sid * NC + cid
    lane = lax.iota(jnp.int32, L)

    pltpu.sync_copy(we_hbm, we_v)
    pltpu.sync_copy(att_hbm, att_v)

    def init_body(i, _):
        amax_priv[pl.ds(16 * i, 16)] = jnp.full((L,), NEG, jnp.float32)
        return 0

    lax.fori_loop(0, AMX // 16, init_body, 0)

    we = [[we_v[k, pl.ds(16 * v, 16)] for v in range(4)] for k in range(4)]
    at = [[att_v[h, pl.ds(16 * v, 16)] for v in range(2)] for h in range(2)]
    ctake = [jnp.full((L,), k, jnp.int32) for k in range(4)]

    def idx_start(c, k):
        base = wid * EPT + CH * c
        pltpu.async_copy(src_hbm.at[pl.ds(base, CH)], si.at[k], sem_i.at[k])
        pltpu.async_copy(dst_hbm.at[pl.ds(base, CH)], di.at[k], sem_i.at[k])
        pltpu.async_copy(ea_hbm.at[pl.ds(4 * base, 4 * CH)],
                         ea_b.at[k, pl.ds(0, 4 * CH)], sem_i.at[k])

    def idx_wait(k):
        pltpu.make_async_copy(src_hbm.at[pl.ds(0, CH)], si.at[k],
                              sem_i.at[k]).wait()
        pltpu.make_async_copy(dst_hbm.at[pl.ds(0, CH)], di.at[k],
                              sem_i.at[k]).wait()
        pltpu.make_async_copy(ea_hbm.at[pl.ds(0, 4 * CH)],
                              ea_b.at[k, pl.ds(0, 4 * CH)],
                              sem_i.at[k]).wait()

    def gather_start(k):
        pltpu.async_copy(xl_hbm.at[si.at[k]],
                         xl_rows.at[pl.ds(CH * k, CH)], sem_g.at[k])
        pltpu.async_copy(xr_hbm.at[di.at[k]],
                         xr_rows.at[pl.ds(CH * k, CH)], sem_g.at[k])

    def gather_wait(k):
        pltpu.make_async_copy(xl_hbm.at[si.at[k]],
                              xl_rows.at[pl.ds(CH * k, CH)],
                              sem_g.at[k]).wait()
        pltpu.make_async_copy(xr_hbm.at[di.at[k]],
                              xr_rows.at[pl.ds(CH * k, CH)],
                              sem_g.at[k]).wait()

    def out_start(c, k):
        base = wid * EPT + CH * c
        pltpu.async_copy(a0_b.at[k], alpha0_hbm.at[pl.ds(base, CH)],
                         sem_o.at[k])
        pltpu.async_copy(a1_b.at[k], alpha1_hbm.at[pl.ds(base, CH)],
                         sem_o.at[k])

    def out_wait(k):
        pltpu.make_async_copy(a0_b.at[k], alpha0_hbm.at[pl.ds(0, CH)],
                              sem_o.at[k]).wait()
        pltpu.make_async_copy(a1_b.at[k], alpha1_hbm.at[pl.ds(0, CH)],
                              sem_o.at[k]).wait()

    idx_start(0, 0)
    idx_start(1, 1)
    idx_wait(0)
    gather_start(0)

    def outer_body(b, _):
        for k in range(RING):
            c = RING * b + k
            k1 = (k + 1) % RING
            k2 = (k + 2) % RING

            @pl.when(c + 2 < NCH)
            def _():
                idx_start(c + 2, k2)

            @pl.when(c + 1 < NCH)
            def _():
                idx_wait(k1)
                gather_start(k1)

            gather_wait(k)

            @pl.when(c >= RING)
            def _():
                out_wait(k)

            def group_body(g, _):
                row0 = CH * k + L * g
                acc0 = jnp.zeros((L,), jnp.float32)
                acc1 = jnp.zeros((L,), jnp.float32)
                for j in range(L):
                    av = ea_b[k, pl.ds(4 * (L * g + j), 16)]
                    t = [jnp.take(av, ctake[q]) for q in range(4)]
                    a0 = jnp.float32(0)
                    a1 = jnp.float32(0)
                    for v in range(4):
                        ef = (t[0] * we[0][v] + t[1] * we[1][v]
                              + t[2] * we[2][v] + t[3] * we[3][v])
                        m = xl_rows[row0 + j, pl.ds(16 * v, 16)] \
                            + xr_rows[row0 + j, pl.ds(16 * v, 16)] + ef
                        lr = jnp.where(m >= 0, m, 0.2 * m)
                        if v < 2:
                            a0 = a0 + jnp.sum(lr * at[0][v], axis=0)
                        else:
                            a1 = a1 + jnp.sum(lr * at[1][v - 2], axis=0)
                    acc0 = jnp.where(lane == j, a0, acc0)
                    acc1 = jnp.where(lane == j, a1, acc1)
                a0_b[k, pl.ds(L * g, L)] = acc0
                a1_b[k, pl.ds(L * g, L)] = acc1

                dvec = di[k, pl.ds(L * g, 16)]
                for h, acc in ((0, acc0), (1, acc1)):
                    kk, vv = plsc.sort_key_val(2 * dvec + h, acc)
                    for sh in (1, 2, 4, 8):
                        sl = jnp.maximum(lane - sh, 0)
                        same = (jnp.take(kk, sl) == kk) & (lane >= sh)
                        vv = jnp.where(same,
                                       jnp.maximum(vv, jnp.take(vv, sl)), vv)
                    nxt = jnp.minimum(lane + 1, L - 1)
                    last = (jnp.take(kk, nxt) != kk) | (lane == L - 1)
                    cur = plsc.load_gather(amax_priv, [kk])
                    plsc.store_scatter(amax_priv, [kk],
                                       jnp.maximum(cur, vv), mask=last)
                return 0

            lax.fori_loop(0, CG, group_body, 0)
            out_start(c, k)
        return 0

    lax.fori_loop(0, NCH // RING, outer_body, 0)
    for k in range(RING):
        out_wait(k)

    pltpu.sync_copy(amax_priv, amax_sh.at[sid])
    plsc.subcore_barrier()

    def comb_start(i, s):
        pltpu.async_copy(amax_sh.at[:, pl.ds(CHK * sid + 64 * i, 64)],
                         xl_rows.at[pl.ds(16 * s, 16)], sem_g.at[s])

    def comb_wait(s):
        pltpu.make_async_copy(amax_sh.at[:, pl.ds(0, 64)],
                              xl_rows.at[pl.ds(16 * s, 16)],
                              sem_g.at[s]).wait()

    comb_start(0, 0)

    def comb_outer(tt, _):
        for s in range(2):
            i = 2 * tt + s
            comb_wait(s)

            @pl.when(i + 1 < CHK // 64)
            def _():
                comb_start(i + 1, 1 - s)

            for v in range(4):
                m = xl_rows[16 * s, pl.ds(16 * v, 16)]
                for r in range(1, NS):
                    m = jnp.maximum(m, xl_rows[16 * s + r, pl.ds(16 * v, 16)])
                amax_priv[pl.ds(64 * i + 16 * v, 16)] = m
        return 0

    lax.fori_loop(0, CHK // 128, comb_outer, 0)
    pltpu.sync_copy(amax_priv.at[pl.ds(0, CHK)],
                    amax_hbm.at[pl.ds(cid * AMX + CHK * sid, CHK)])


def _sc_alpha(src, dst, ea_flat, xl, xr, W_e, att):
    return pl.kernel(
        _sc_alpha_body,
        out_type=[
            jax.ShapeDtypeStruct((E,), jnp.float32),
            jax.ShapeDtypeStruct((E,), jnp.float32),
            jax.ShapeDtypeStruct((NC * AMX,), jnp.float32),
        ],
        mesh=_mesh,
        scratch_types=[
            pltpu.VMEM((RING, CH), jnp.int32),
            pltpu.VMEM((RING, CH), jnp.int32),
            pltpu.VMEM((RING, 4 * CH + 16), jnp.float32),
            pltpu.VMEM((RING * CH, D), jnp.float32),
            pltpu.VMEM((RING * CH, D), jnp.float32),
            pltpu.VMEM((RING, CH), jnp.float32),
            pltpu.VMEM((RING, CH), jnp.float32),
            pltpu.VMEM((AMX,), jnp.float32),
            pltpu.VMEM((4, D), jnp.float32),
            pltpu.VMEM((2, 32), jnp.float32),
            pltpu.VMEM_SHARED((NS, AMX), jnp.float32),
            pltpu.SemaphoreType.DMA((RING,)),
            pltpu.SemaphoreType.DMA((RING,)),
            pltpu.SemaphoreType.DMA((RING,)),
        ],
        compiler_params=_sc_params,
    )(src, dst, ea_flat, xl, xr, W_e, att)


def _combine_body(amaxp_ref, xl_ref, xr_ref, efm_ref, att_ref,
                  amax_ref, wself_ref):
    partmax = jnp.max(amaxp_ref[...], axis=0)
    m = xl_ref[...] + xr_ref[...] + efm_ref[...]
    lr = jnp.where(m >= 0, m, 0.2 * m)
    a0 = jnp.sum(lr[:, 0:32] * att_ref[0:1, :], axis=1, keepdims=True)
    a1 = jnp.sum(lr[:, 32:64] * att_ref[1:2, :], axis=1, keepdims=True)
    aself = jnp.concatenate([a0, a1], axis=1)
    aself = jnp.concatenate(
        [aself, jnp.full((NPAD - N, 2), NEG, jnp.float32)], axis=0)
    amax = jnp.maximum(partmax, aself)
    amax_ref[...] = amax
    wself_ref[...] = jnp.exp(aself - amax)


def _combine(amax_part, xl, xr, ef_mean, att):
    return pl.pallas_call(
        _combine_body,
        out_shape=[
            jax.ShapeDtypeStruct((NPAD, 2), jnp.float32),
            jax.ShapeDtypeStruct((NPAD, 2), jnp.float32),
        ],
        compiler_params=pltpu.CompilerParams(vmem_limit_bytes=100 * 1024 * 1024),
    )(amax_part, xl, xr, ef_mean, att)


def _sc_scatter_body(src_hbm, dst_hbm, alpha0_hbm, alpha1_hbm, amax_hbm,
                     xl_hbm, numer_hbm,
                     si, di, xl_rows, msg_rows, a0_b, a1_b, amax_loc,
                     numer_sp, sem_i, sem_g, sem_sc):
    cid = lax.axis_index("c")
    sid = lax.axis_index("s")
    wid = sid * NC + cid
    lane = lax.iota(jnp.int32, L)
    cj = [jnp.full((L,), j, jnp.int32) for j in range(L)]

    def zero_body(r, _):
        for v in range(5):
            msg_rows[r, pl.ds(16 * v, 16)] = jnp.zeros((L,), jnp.float32)
        return 0

    lax.fori_loop(0, BLK, zero_body, 0)
    npt = N // NS
    pltpu.sync_copy(msg_rows.at[pl.ds(0, BLK)],
                    numer_sp.at[pl.ds(npt * sid, BLK)])
    pltpu.sync_copy(msg_rows.at[pl.ds(0, npt - BLK)],
                    numer_sp.at[pl.ds(npt * sid + BLK, npt - BLK)])
    pltpu.sync_copy(amax_hbm, amax_loc)
    plsc.subcore_barrier()

    def idx_start(c, k):
        base = wid * EPT + CH * c
        pltpu.async_copy(src_hbm.at[pl.ds(base, CH)], si.at[k], sem_i.at[k])
        pltpu.async_copy(dst_hbm.at[pl.ds(base, CH)], di.at[k], sem_i.at[k])
        pltpu.async_copy(alpha0_hbm.at[pl.ds(base, CH)], a0_b.at[k],
                         sem_i.at[k])
        pltpu.async_copy(alpha1_hbm.at[pl.ds(base, CH)], a1_b.at[k],
                         sem_i.at[k])

    def idx_wait(k):
        pltpu.make_async_copy(src_hbm.at[pl.ds(0, CH)], si.at[k],
                              sem_i.at[k]).wait()
        pltpu.make_async_copy(dst_hbm.at[pl.ds(0, CH)], di.at[k],
                              sem_i.at[k]).wait()
        pltpu.make_async_copy(alpha0_hbm.at[pl.ds(0, CH)], a0_b.at[k],
                              sem_i.at[k]).wait()
        pltpu.make_async_copy(alpha1_hbm.at[pl.ds(0, CH)], a1_b.at[k],
                              sem_i.at[k]).wait()

    def gather_start(k):
        pltpu.async_copy(xl_hbm.at[si.at[k]],
                         xl_rows.at[pl.ds(CH * k, CH)], sem_g.at[k])

    def gather_wait(k):
        pltpu.make_async_copy(xl_hbm.at[si.at[k]],
                              xl_rows.at[pl.ds(CH * k, CH)],
                              sem_g.at[k]).wait()

    def scatter_start(k):
        pltpu.async_copy(msg_rows.at[pl.ds(CH * k, CH)],
                         numer_sp.at[di.at[k]], sem_sc.at[k], add=True)

    def scatter_wait(k):
        pltpu.make_async_copy(msg_rows.at[pl.ds(CH * k, CH)],
                              numer_sp.at[di.at[k]], sem_sc.at[k]).wait()

    idx_start(0, 0)
    idx_start(1, 1)
    idx_wait(0)
    gather_start(0)

    def outer_body(b, _):
        for k in range(RING):
            c = RING * b + k
            k1 = (k + 1) % RING
            k2 = (k + 2) % RING

            @pl.when(c + 2 < NCH)
            def _():
                @pl.when(c >= 3)
                def _():
                    scatter_wait(k2)

                idx_start(c + 2, k2)

            @pl.when(c + 1 < NCH)
            def _():
                idx_wait(k1)
                gather_start(k1)

            gather_wait(k)

            def group_body(g, _):
                row0 = CH * k + L * g
                dvec = di[k, pl.ds(L * g, 16)]
                am0 = plsc.load_gather(amax_loc, [2 * dvec])
                am1 = plsc.load_gather(amax_loc, [2 * dvec + 1])
                w0 = jnp.exp(a0_b[k, pl.ds(L * g, L)] - am0)
                w1 = jnp.exp(a1_b[k, pl.ds(L * g, L)] - am1)
                for j in range(L):
                    e = row0 + j
                    w0j = jnp.take(w0, cj[j])
                    w1j = jnp.take(w1, cj[j])
                    msg_rows[e, pl.ds(0, 16)] = xl_rows[e, pl.ds(0, 16)] * w0j
                    msg_rows[e, pl.ds(16, 16)] = \
                        xl_rows[e, pl.ds(16, 16)] * w0j
                    msg_rows[e, pl.ds(32, 16)] = \
                        xl_rows[e, pl.ds(32, 16)] * w1j
                    msg_rows[e, pl.ds(48, 16)] = \
                        xl_rows[e, pl.ds(48, 16)] * w1j
                    den = jnp.where(lane == 0, w0j,
                                    jnp.where(lane == 1, w1j, 0.0))
                    msg_rows[e, pl.ds(64, 16)] = den
                return 0

            lax.fori_loop(0, CG, group_body, 0)
            scatter_start(k)
        return 0

    lax.fori_loop(0, NCH // RING, outer_body, 0)
    for k in range(RING):
        scatter_wait(k)
    plsc.subcore_barrier()

    @pl.when(sid == 0)
    def _():
        pltpu.sync_copy(numer_sp, numer_hbm.at[pl.ds(cid * N, N)])


def _sc_scatter(src, dst, alpha0, alpha1, amax_flat, xl):
    return pl.kernel(
        _sc_scatter_body,
        out_type=jax.ShapeDtypeStruct((NC * N, 80), jnp.float32),
        mesh=_mesh,
        scratch_types=[
            pltpu.VMEM((RING, CH), jnp.int32),
            pltpu.VMEM((RING, CH), jnp.int32),
            pltpu.VMEM((RING * CH, D), jnp.float32),
            pltpu.VMEM((RING * CH, 80), jnp.float32),
            pltpu.VMEM((RING, CH), jnp.float32),
            pltpu.VMEM((RING, CH), jnp.float32),
            pltpu.VMEM((AMX,), jnp.float32),
            pltpu.VMEM_SHARED((N, 80), jnp.float32),
            pltpu.SemaphoreType.DMA((RING,)),
            pltpu.SemaphoreType.DMA((RING,)),
            pltpu.SemaphoreType.DMA((RING,)),
        ],
        compiler_params=_sc_params,
    )(src, dst, alpha0, alpha1, amax_flat, xl)


def _final_body(numer_ref, xl_ref, wself_ref, bias_ref, wout_ref, bout_ref,
                z_ref):
    num = numer_ref[0:N, :] + numer_ref[N:2 * N, :]
    w = wself_ref[0:N, :]
    w0 = w[:, 0:1]
    w1 = w[:, 1:2]
    xlv = xl_ref[...]
    n0 = num[:, 0:32] + w0 * xlv[:, 0:32]
    n1 = num[:, 32:64] + w1 * xlv[:, 32:64]
    d0 = num[:, 64:65] + w0 + 1e-16
    d1 = num[:, 65:66] + w1 + 1e-16
    g = jnp.concatenate([n0 / d0, n1 / d1], axis=1) + bias_ref[...]
    hh = jnp.where(g > 0, g, jnp.exp(g) - 1.0)
    z_ref[...] = jnp.dot(hh, wout_ref[...]) + bout_ref[...]


def _final(numer_part, xl, wself, bias_conv, W_out, b_out):
    return pl.pallas_call(
        _final_body,
        out_shape=jax.ShapeDtypeStruct((N, OUT_DIM), jnp.float32),
        compiler_params=pltpu.CompilerParams(vmem_limit_bytes=100 * 1024 * 1024),
    )(numer_part, xl, wself, bias_conv, W_out, b_out)


def kernel(x, edge_index, edge_attr, W_in, b_in, W_l, b_l, W_r, b_r, W_e,
           att, bias_conv, W_out, b_out):
    src = edge_index[0]
    dst = edge_index[1]
    ea_flat = edge_attr.reshape(-1)
    ea2 = edge_attr.reshape(E // 128, 512)
    F = jnp.tile(W_e, (128, 1))

    xl, xr, ef_mean = _proj(x, W_in, b_in.reshape(1, D), W_l,
                            b_l.reshape(1, D), W_r, b_r.reshape(1, D),
                            ea2, F)
    alpha0, alpha1, amax_sc = _sc_alpha(src, dst, ea_flat, xl, xr, W_e, att)
    amax, wself = _combine(amax_sc.reshape(NC, NPAD, 2), xl, xr,
                           ef_mean, att)
    numer_part = _sc_scatter(src, dst, alpha0, alpha1, amax.reshape(AMX), xl)
    z = _final(numer_part, xl, wself, bias_conv.reshape(1, D), W_out,
               b_out.reshape(1, OUT_DIM))
    return z


# speedup vs baseline: 80.5238x; 1.7041x over previous
import jax
import jax.numpy as jnp
from jax import lax
from jax.experimental import pallas as pl
from jax.experimental.pallas import tpu as pltpu
from jax.experimental.pallas import tpu_sc as plsc

N = 10000
E = 320000
IN_DIM = 128
D = 64
OUT_DIM = 48
NC, NS, L = 2, 16, 16
NW = NC * NS
EPT = E // NW
NPAD = 10240
AMX = 2 * NPAD
CHK = AMX // NS
NEG = -1e30
RING = 5
CH = 80
NCH = EPT // CH
CG = CH // L

_mesh = plsc.VectorSubcoreMesh(
    core_axis_name="c", subcore_axis_name="s", num_cores=NC, num_subcores=NS
)
_sc_params = pltpu.CompilerParams(
    needs_layout_passes=False, use_tc_tiling_on_sc=False
)


def _proj_body(x_ref, win_ref, bin_ref, wl_ref, bl_ref, wr_ref, br_ref,
               xl_ref, xr_ref):
    h = jnp.dot(x_ref[...], win_ref[...]) + bin_ref[...]
    h = jnp.where(h > 0, h, jnp.exp(h) - 1.0)
    xl_ref[...] = jnp.dot(h, wl_ref[...]) + bl_ref[...]
    xr_ref[...] = jnp.dot(h, wr_ref[...]) + br_ref[...]


def _proj(x, W_in, b_in, W_l, b_l, W_r, b_r):
    return pl.pallas_call(
        _proj_body,
        out_shape=[
            jax.ShapeDtypeStruct((N, D), jnp.float32),
            jax.ShapeDtypeStruct((N, D), jnp.float32),
        ],
        compiler_params=pltpu.CompilerParams(vmem_limit_bytes=100 * 1024 * 1024),
    )(x, W_in, b_in, W_l, b_l, W_r, b_r)


def _sc_alpha_body(ei_hbm, ea_hbm, xl_hbm, xr_hbm, we_hbm, att_hbm,
                   alpha0_hbm, alpha1_hbm, amax_hbm, easum_hbm,
                   ei_b, ea_b, xl_rows, xr_rows, a0_b, a1_b, amax_priv,
                   easum_v, we_v, att_v, amax_sh, sem_i, sem_g, sem_o):
    cid = lax.axis_index("c")
    sid = lax.axis_index("s")
    wid = sid * NC + cid
    lane = lax.iota(jnp.int32, L)

    pltpu.sync_copy(we_hbm, we_v)
    pltpu.sync_copy(att_hbm, att_v)

    def init_body(i, _):
        amax_priv[pl.ds(16 * i, 16)] = jnp.full((L,), NEG, jnp.float32)
        return 0

    lax.fori_loop(0, AMX // 16, init_body, 0)
    for q in range(4):
        easum_v[pl.ds(16 * q, 16)] = jnp.zeros((L,), jnp.float32)

    we = [[we_v[k, pl.ds(16 * v, 16)] for v in range(4)] for k in range(4)]
    at = [[att_v[h, pl.ds(16 * v, 16)] for v in range(2)] for h in range(2)]
    ctake = [jnp.full((L,), j, jnp.int32) for j in range(L)]

    def idx_start(c, k):
        base = wid * EPT + CH * c
        pltpu.async_copy(ei_hbm.at[:, pl.ds(base, CH)], ei_b.at[k],
                         sem_i.at[k])
        pltpu.async_copy(ea_hbm.at[:, pl.ds(base, CH)], ea_b.at[k],
                         sem_i.at[k])

    def idx_wait(k):
        pltpu.make_async_copy(ei_hbm.at[:, pl.ds(0, CH)], ei_b.at[k],
                              sem_i.at[k]).wait()
        pltpu.make_async_copy(ea_hbm.at[:, pl.ds(0, CH)], ea_b.at[k],
                              sem_i.at[k]).wait()

    def gather_start(k):
        pltpu.async_copy(xl_hbm.at[ei_b.at[k, 0]],
                         xl_rows.at[pl.ds(CH * k, CH)], sem_g.at[k])
        pltpu.async_copy(xr_hbm.at[ei_b.at[k, 1]],
                         xr_rows.at[pl.ds(CH * k, CH)], sem_g.at[k])

    def gather_wait(k):
        pltpu.make_async_copy(xl_hbm.at[ei_b.at[k, 0]],
                              xl_rows.at[pl.ds(CH * k, CH)],
                              sem_g.at[k]).wait()
        pltpu.make_async_copy(xr_hbm.at[ei_b.at[k, 1]],
                              xr_rows.at[pl.ds(CH * k, CH)],
                              sem_g.at[k]).wait()

    def out_start(c, k):
        base = wid * EPT + CH * c
        pltpu.async_copy(a0_b.at[k], alpha0_hbm.at[pl.ds(base, CH)],
                         sem_o.at[k])
        pltpu.async_copy(a1_b.at[k], alpha1_hbm.at[pl.ds(base, CH)],
                         sem_o.at[k])

    def out_wait(k):
        pltpu.make_async_copy(a0_b.at[k], alpha0_hbm.at[pl.ds(0, CH)],
                              sem_o.at[k]).wait()
        pltpu.make_async_copy(a1_b.at[k], alpha1_hbm.at[pl.ds(0, CH)],
                              sem_o.at[k]).wait()

    idx_start(0, 0)
    idx_start(1, 1)
    idx_wait(0)
    gather_start(0)

    def outer_body(b, _):
        for k in range(RING):
            c = RING * b + k
            k1 = (k + 1) % RING
            k2 = (k + 2) % RING

            @pl.when(c + 2 < NCH)
            def _():
                idx_start(c + 2, k2)

            @pl.when(c + 1 < NCH)
            def _():
                idx_wait(k1)
                gather_start(k1)

            gather_wait(k)

            @pl.when(c >= RING)
            def _():
                out_wait(k)

            def group_body(g, _):
                row0 = CH * k + L * g
                ea = [ea_b[k, q, pl.ds(L * g, 16)] for q in range(4)]
                for q in range(4):
                    easum_v[pl.ds(16 * q, 16)] = \
                        easum_v[pl.ds(16 * q, 16)] + ea[q]
                acc0 = jnp.zeros((L,), jnp.float32)
                acc1 = jnp.zeros((L,), jnp.float32)
                for j in range(L):
                    t = [jnp.take(ea[q], ctake[j]) for q in range(4)]
                    a0 = jnp.float32(0)
                    a1 = jnp.float32(0)
                    for v in range(4):
                        ef = (t[0] * we[0][v] + t[1] * we[1][v]
                              + t[2] * we[2][v] + t[3] * we[3][v])
                        m = xl_rows[row0 + j, pl.ds(16 * v, 16)] \
                            + xr_rows[row0 + j, pl.ds(16 * v, 16)] + ef
                        lr = jnp.maximum(m, 0.2 * m)
                        if v < 2:
                            a0 = a0 + jnp.sum(lr * at[0][v], axis=0)
                        else:
                            a1 = a1 + jnp.sum(lr * at[1][v - 2], axis=0)
                    acc0 = jnp.where(lane == j, a0, acc0)
                    acc1 = jnp.where(lane == j, a1, acc1)
                a0_b[k, pl.ds(L * g, L)] = acc0
                a1_b[k, pl.ds(L * g, L)] = acc1

                dvec = ei_b[k, 1, pl.ds(L * g, 16)]
                for h, acc in ((0, acc0), (1, acc1)):
                    kk, vv = plsc.sort_key_val(2 * dvec + h, acc)
                    for sh in (1, 2, 4, 8):
                        sl = jnp.maximum(lane - sh, 0)
                        same = (jnp.take(kk, sl) == kk) & (lane >= sh)
                        vv = jnp.where(same,
                                       jnp.maximum(vv, jnp.take(vv, sl)), vv)
                    nxt = jnp.minimum(lane + 1, L - 1)
                    last = (jnp.take(kk, nxt) != kk) | (lane == L - 1)
                    cur = plsc.load_gather(amax_priv, [kk])
                    plsc.store_scatter(amax_priv, [kk],
                                       jnp.maximum(cur, vv), mask=last)
                return 0

            lax.fori_loop(0, CG, group_body, 0)
            out_start(c, k)
        return 0

    lax.fori_loop(0, NCH // RING, outer_body, 0)
    for k in range(RING):
        out_wait(k)
    pltpu.sync_copy(easum_v, easum_hbm.at[pl.ds(64 * wid, 64)])

    pltpu.sync_copy(amax_priv, amax_sh.at[sid])
    plsc.subcore_barrier()

    def comb_start(i, s):
        pltpu.async_copy(amax_sh.at[:, pl.ds(CHK * sid + 64 * i, 64)],
                         xl_rows.at[pl.ds(16 * s, 16)], sem_g.at[s])

    def comb_wait(s):
        pltpu.make_async_copy(amax_sh.at[:, pl.ds(0, 64)],
                              xl_rows.at[pl.ds(16 * s, 16)],
                              sem_g.at[s]).wait()

    comb_start(0, 0)

    def comb_outer(tt, _):
        for s in range(2):
            i = 2 * tt + s
            comb_wait(s)

            @pl.when(i + 1 < CHK // 64)
            def _():
                comb_start(i + 1, 1 - s)

            for v in range(4):
                m = xl_rows[16 * s, pl.ds(16 * v, 16)]
                for r in range(1, NS):
                    m = jnp.maximum(m, xl_rows[16 * s + r, pl.ds(16 * v, 16)])
                amax_priv[pl.ds(64 * i + 16 * v, 16)] = m
        return 0

    lax.fori_loop(0, CHK // 128, comb_outer, 0)
    pltpu.sync_copy(amax_priv.at[pl.ds(0, CHK)],
                    amax_hbm.at[pl.ds(cid * AMX + CHK * sid, CHK)])


def _sc_alpha(ei, ea_T, xl, xr, W_e, att):
    return pl.kernel(
        _sc_alpha_body,
        name="sc_alpha",
        out_type=[
            jax.ShapeDtypeStruct((E,), jnp.float32),
            jax.ShapeDtypeStruct((E,), jnp.float32),
            jax.ShapeDtypeStruct((NC * AMX,), jnp.float32),
            jax.ShapeDtypeStruct((NW * 64,), jnp.float32),
        ],
        mesh=_mesh,
        scratch_types=[
            pltpu.VMEM((RING, 2, CH), jnp.int32),
            pltpu.VMEM((RING, 4, CH), jnp.float32),
            pltpu.VMEM((RING * CH, D), jnp.float32),
            pltpu.VMEM((RING * CH, D), jnp.float32),
            pltpu.VMEM((RING, CH), jnp.float32),
            pltpu.VMEM((RING, CH), jnp.float32),
            pltpu.VMEM((AMX,), jnp.float32),
            pltpu.VMEM((64,), jnp.float32),
            pltpu.VMEM((4, D), jnp.float32),
            pltpu.VMEM((2, 32), jnp.float32),
            pltpu.VMEM_SHARED((NS, AMX), jnp.float32),
            pltpu.SemaphoreType.DMA((RING,)),
            pltpu.SemaphoreType.DMA((RING,)),
            pltpu.SemaphoreType.DMA((RING,)),
        ],
        compiler_params=_sc_params,
    )(ei, ea_T, xl, xr, W_e, att)


def _sc_scatter_body(ei_hbm, alpha0_hbm, alpha1_hbm, amax_hbm, xl_hbm,
                     numer_hbm,
                     ei_b, xl_rows, msg_rows, a0_b, a1_b, amax_loc, tmp,
                     numer_sp, sem_i, sem_g, sem_sc):
    cid = lax.axis_index("c")
    sid = lax.axis_index("s")
    wid = sid * NC + cid
    lane = lax.iota(jnp.int32, L)
    cj = [jnp.full((L,), j, jnp.int32) for j in range(L)]

    def zero_body(r, _):
        for v in range(5):
            msg_rows[r, pl.ds(16 * v, 16)] = jnp.zeros((L,), jnp.float32)
        return 0

    lax.fori_loop(0, RING * CH, zero_body, 0)
    npt = N // NS
    pltpu.sync_copy(msg_rows.at[pl.ds(0, 400)],
                    numer_sp.at[pl.ds(npt * sid, 400)])
    pltpu.sync_copy(msg_rows.at[pl.ds(0, npt - 400)],
                    numer_sp.at[pl.ds(npt * sid + 400, npt - 400)])

    pltpu.sync_copy(amax_hbm.at[pl.ds(0, AMX)], amax_loc)
    for p in range(AMX // CHK):
        pltpu.sync_copy(amax_hbm.at[pl.ds(AMX + CHK * p, CHK)], tmp)

        def mx_body(i, _, _p=p):
            amax_loc[pl.ds(CHK * _p + 16 * i, 16)] = jnp.maximum(
                amax_loc[pl.ds(CHK * _p + 16 * i, 16)], tmp[pl.ds(16 * i, 16)])
            return 0

        lax.fori_loop(0, CHK // 16, mx_body, 0)
    plsc.subcore_barrier()

    def idx_start(c, k):
        base = wid * EPT + CH * c
        pltpu.async_copy(ei_hbm.at[:, pl.ds(base, CH)], ei_b.at[k],
                         sem_i.at[k])
        pltpu.async_copy(alpha0_hbm.at[pl.ds(base, CH)], a0_b.at[k],
                         sem_i.at[k])
        pltpu.async_copy(alpha1_hbm.at[pl.ds(base, CH)], a1_b.at[k],
                         sem_i.at[k])

    def idx_wait(k):
        pltpu.make_async_copy(ei_hbm.at[:, pl.ds(0, CH)], ei_b.at[k],
                              sem_i.at[k]).wait()
        pltpu.make_async_copy(alpha0_hbm.at[pl.ds(0, CH)], a0_b.at[k],
                              sem_i.at[k]).wait()
        pltpu.make_async_copy(alpha1_hbm.at[pl.ds(0, CH)], a1_b.at[k],
                              sem_i.at[k]).wait()

    def gather_start(k):
        pltpu.async_copy(xl_hbm.at[ei_b.at[k, 0]],
                         xl_rows.at[pl.ds(CH * k, CH)], sem_g.at[k])

    def gather_wait(k):
        pltpu.make_async_copy(xl_hbm.at[ei_b.at[k, 0]],
                              xl_rows.at[pl.ds(CH * k, CH)],
                              sem_g.at[k]).wait()

    def scatter_start(k):
        pltpu.async_copy(msg_rows.at[pl.ds(CH * k, CH)],
                         numer_sp.at[ei_b.at[k, 1]], sem_sc.at[k], add=True)

    def scatter_wait(k):
        pltpu.make_async_copy(msg_rows.at[pl.ds(CH * k, CH)],
                              numer_sp.at[ei_b.at[k, 1]],
                              sem_sc.at[k]).wait()

    idx_start(0, 0)
    idx_start(1, 1)
    idx_wait(0)
    gather_start(0)

    def outer_body(b, _):
        for k in range(RING):
            c = RING * b + k
            k1 = (k + 1) % RING
            k2 = (k + 2) % RING

            @pl.when(c + 2 < NCH)
            def _():
                @pl.when(c >= 3)
                def _():
                    scatter_wait(k2)

                idx_start(c + 2, k2)

            @pl.when(c + 1 < NCH)
            def _():
                idx_wait(k1)
                gather_start(k1)

            gather_wait(k)

            def group_body(g, _):
                row0 = CH * k + L * g
                dvec = ei_b[k, 1, pl.ds(L * g, 16)]
                am0 = plsc.load_gather(amax_loc, [2 * dvec])
                am1 = plsc.load_gather(amax_loc, [2 * dvec + 1])
                w0 = jnp.exp(a0_b[k, pl.ds(L * g, L)] - am0)
                w1 = jnp.exp(a1_b[k, pl.ds(L * g, L)] - am1)
                for j in range(L):
                    e = row0 + j
                    w0j = jnp.take(w0, cj[j])
                    w1j = jnp.take(w1, cj[j])
                    msg_rows[e, pl.ds(0, 16)] = xl_rows[e, pl.ds(0, 16)] * w0j
                    msg_rows[e, pl.ds(16, 16)] = \
                        xl_rows[e, pl.ds(16, 16)] * w0j
                    msg_rows[e, pl.ds(32, 16)] = \
                        xl_rows[e, pl.ds(32, 16)] * w1j
                    msg_rows[e, pl.ds(48, 16)] = \
                        xl_rows[e, pl.ds(48, 16)] * w1j
                    den = jnp.where(lane == 0, w0j,
                                    jnp.where(lane == 1, w1j, 0.0))
                    msg_rows[e, pl.ds(64, 16)] = den
                return 0

            lax.fori_loop(0, CG, group_body, 0)
            scatter_start(k)
        return 0

    lax.fori_loop(0, NCH // RING, outer_body, 0)
    for k in range(RING):
        scatter_wait(k)
    plsc.subcore_barrier()

    @pl.when(sid == 0)
    def _():
        pltpu.sync_copy(numer_sp, numer_hbm.at[pl.ds(cid * N, N)])


def _sc_scatter(ei, alpha0, alpha1, amax_sc, xl):
    return pl.kernel(
        _sc_scatter_body,
        name="sc_scatter",
        out_type=jax.ShapeDtypeStruct((NC * N, 80), jnp.float32),
        mesh=_mesh,
        scratch_types=[
            pltpu.VMEM((RING, 2, CH), jnp.int32),
            pltpu.VMEM((RING * CH, D), jnp.float32),
            pltpu.VMEM((RING * CH, 80), jnp.float32),
            pltpu.VMEM((RING, CH), jnp.float32),
            pltpu.VMEM((RING, CH), jnp.float32),
            pltpu.VMEM((AMX,), jnp.float32),
            pltpu.VMEM((CHK,), jnp.float32),
            pltpu.VMEM_SHARED((N, 80), jnp.float32),
            pltpu.SemaphoreType.DMA((RING,)),
            pltpu.SemaphoreType.DMA((RING,)),
            pltpu.SemaphoreType.DMA((RING,)),
        ],
        compiler_params=_sc_params,
    )(ei, alpha0, alpha1, amax_sc, xl)


def _final_body(numer_ref, xl_ref, xr_ref, amaxsc_ref, easum_ref, fsel_ref,
                att_ref, bias_ref, wout_ref, bout_ref, z_ref):
    num = numer_ref[0] + numer_ref[1]
    pm = jnp.maximum(amaxsc_ref[0], amaxsc_ref[1])

    s1 = jnp.sum(easum_ref[...], axis=0, keepdims=True)
    efm = jnp.dot(s1, fsel_ref[...]) * (1.0 / E)
    m = xl_ref[...] + xr_ref[...] + efm
    lr = jnp.where(m >= 0, m, 0.2 * m)
    a0 = jnp.sum(lr[:, 0:32] * att_ref[0:1, :], axis=1, keepdims=True)
    a1 = jnp.sum(lr[:, 32:64] * att_ref[1:2, :], axis=1, keepdims=True)
    aself = jnp.concatenate([a0, a1], axis=1)

    amax = jnp.maximum(pm, aself)
    sc = jnp.exp(pm - amax)
    w = jnp.exp(aself - amax)
    sc0 = sc[:, 0:1]
    sc1 = sc[:, 1:2]
    w0 = w[:, 0:1]
    w1 = w[:, 1:2]
    xlv = xl_ref[...]
    n0 = num[:, 0:32] * sc0 + w0 * xlv[:, 0:32]
    n1 = num[:, 32:64] * sc1 + w1 * xlv[:, 32:64]
    d0 = num[:, 64:65] * sc0 + w0 + 1e-16
    d1 = num[:, 65:66] * sc1 + w1 + 1e-16
    g = jnp.concatenate([n0 / d0, n1 / d1], axis=1) + bias_ref[...]
    hh = jnp.where(g > 0, g, jnp.exp(g) - 1.0)
    z_ref[...] = jnp.dot(hh, wout_ref[...]) + bout_ref[...]


BN = 1000


def _final(numer3, xl, xr, amax_sc, easum, Fsel, att, bias_conv, W_out,
           b_out):
    return pl.pallas_call(
        _final_body,
        grid=(N // BN,),
        in_specs=[
            pl.BlockSpec((NC, BN, 80), lambda i: (0, i, 0)),
            pl.BlockSpec((BN, D), lambda i: (i, 0)),
            pl.BlockSpec((BN, D), lambda i: (i, 0)),
            pl.BlockSpec((NC, BN, 2), lambda i: (0, i, 0)),
            pl.BlockSpec((NW, 64), lambda i: (0, 0)),
            pl.BlockSpec((64, D), lambda i: (0, 0)),
            pl.BlockSpec((2, 32), lambda i: (0, 0)),
            pl.BlockSpec((1, D), lambda i: (0, 0)),
            pl.BlockSpec((D, OUT_DIM), lambda i: (0, 0)),
            pl.BlockSpec((1, OUT_DIM), lambda i: (0, 0)),
        ],
        out_specs=pl.BlockSpec((BN, OUT_DIM), lambda i: (i, 0)),
        out_shape=jax.ShapeDtypeStruct((N, OUT_DIM), jnp.float32),
        compiler_params=pltpu.CompilerParams(vmem_limit_bytes=100 * 1024 * 1024),
    )(numer3, xl, xr, amax_sc, easum, Fsel, att, bias_conv, W_out, b_out)


def kernel(x, edge_index, edge_attr, W_in, b_in, W_l, b_l, W_r, b_r, W_e,
           att, bias_conv, W_out, b_out):
    ea_T = edge_attr.T
    Fsel = jnp.repeat(W_e, 16, axis=0)

    xl, xr = _proj(x, W_in, b_in.reshape(1, D), W_l, b_l.reshape(1, D),
                   W_r, b_r.reshape(1, D))
    alpha0, alpha1, amax_sc, easum = _sc_alpha(edge_index, ea_T, xl, xr,
                                               W_e, att)
    numer_part = _sc_scatter(edge_index, alpha0, alpha1, amax_sc, xl)
    z = _final(numer_part.reshape(NC, N, 80), xl, xr,
               amax_sc.reshape(NC, NPAD, 2), easum.reshape(NW, 64), Fsel,
               att, bias_conv.reshape(1, D), W_out,
               b_out.reshape(1, OUT_DIM))
    return z


# speedup vs baseline: 82.5189x; 1.0248x over previous
import jax
import jax.numpy as jnp
from jax import lax
from jax.experimental import pallas as pl
from jax.experimental.pallas import tpu as pltpu
from jax.experimental.pallas import tpu_sc as plsc

N = 10000
E = 320000
IN_DIM = 128
D = 64
OUT_DIM = 48
NC, NS, L = 2, 16, 16
NW = NC * NS
EPT = E // NW
NPAD = 10240
AMX = 2 * NPAD
CHK = AMX // NS
NEG = -1e30
RING = 5
CH = 80
NCH = EPT // CH
CG = CH // L

_mesh = plsc.VectorSubcoreMesh(
    core_axis_name="c", subcore_axis_name="s", num_cores=NC, num_subcores=NS
)
_sc_params = pltpu.CompilerParams(
    needs_layout_passes=False, use_tc_tiling_on_sc=False
)


def _proj_body(x_ref, win_ref, bin_ref, wl_ref, bl_ref, wr_ref, br_ref,
               xl_ref, xr_ref):
    h = jnp.dot(x_ref[...], win_ref[...]) + bin_ref[...]
    h = jnp.where(h > 0, h, jnp.exp(h) - 1.0)
    xl_ref[...] = jnp.dot(h, wl_ref[...]) + bl_ref[...]
    xr_ref[...] = jnp.dot(h, wr_ref[...]) + br_ref[...]


def _proj(x, W_in, b_in, W_l, b_l, W_r, b_r):
    return pl.pallas_call(
        _proj_body,
        out_shape=[
            jax.ShapeDtypeStruct((N, D), jnp.float32),
            jax.ShapeDtypeStruct((N, D), jnp.float32),
        ],
        compiler_params=pltpu.CompilerParams(vmem_limit_bytes=100 * 1024 * 1024),
    )(x, W_in, b_in, W_l, b_l, W_r, b_r)


def _sc_alpha_body(ei_hbm, ea_hbm, xl_hbm, xr_hbm, we_hbm, att_hbm,
                   alpha0_hbm, alpha1_hbm, amax_hbm, easum_hbm,
                   ei_b, ea_b, xl_rows, xr_rows, a0_b, a1_b, amax_priv,
                   easum_v, we_v, att_v, amax_sh, sem_i, sem_g, sem_o):
    cid = lax.axis_index("c")
    sid = lax.axis_index("s")
    wid = sid * NC + cid
    lane = lax.iota(jnp.int32, L)

    pltpu.sync_copy(we_hbm, we_v)
    pltpu.sync_copy(att_hbm, att_v)

    def init_body(i, _):
        amax_priv[pl.ds(16 * i, 16)] = jnp.full((L,), NEG, jnp.float32)
        return 0

    lax.fori_loop(0, AMX // 16, init_body, 0)
    for q in range(4):
        easum_v[pl.ds(16 * q, 16)] = jnp.zeros((L,), jnp.float32)

    we = [[we_v[k, pl.ds(16 * v, 16)] for v in range(4)] for k in range(4)]
    at = [[att_v[h, pl.ds(16 * v, 16)] for v in range(2)] for h in range(2)]
    ctake = [jnp.full((L,), j, jnp.int32) for j in range(L)]

    def idx_start(c, k):
        base = wid * EPT + CH * c
        pltpu.async_copy(ei_hbm.at[:, pl.ds(base, CH)], ei_b.at[k],
                         sem_i.at[k])
        pltpu.async_copy(ea_hbm.at[:, pl.ds(base, CH)], ea_b.at[k],
                         sem_i.at[k])

    def idx_wait(k):
        pltpu.make_async_copy(ei_hbm.at[:, pl.ds(0, CH)], ei_b.at[k],
                              sem_i.at[k]).wait()
        pltpu.make_async_copy(ea_hbm.at[:, pl.ds(0, CH)], ea_b.at[k],
                              sem_i.at[k]).wait()

    def gather_start(k):
        pltpu.async_copy(xl_hbm.at[ei_b.at[k, 0]],
                         xl_rows.at[pl.ds(CH * k, CH)], sem_g.at[k])
        pltpu.async_copy(xr_hbm.at[ei_b.at[k, 1]],
                         xr_rows.at[pl.ds(CH * k, CH)], sem_g.at[k])

    def gather_wait(k):
        pltpu.make_async_copy(xl_hbm.at[ei_b.at[k, 0]],
                              xl_rows.at[pl.ds(CH * k, CH)],
                              sem_g.at[k]).wait()
        pltpu.make_async_copy(xr_hbm.at[ei_b.at[k, 1]],
                              xr_rows.at[pl.ds(CH * k, CH)],
                              sem_g.at[k]).wait()

    def out_start(c, k):
        base = wid * EPT + CH * c
        pltpu.async_copy(a0_b.at[k], alpha0_hbm.at[pl.ds(base, CH)],
                         sem_o.at[k])
        pltpu.async_copy(a1_b.at[k], alpha1_hbm.at[pl.ds(base, CH)],
                         sem_o.at[k])

    def out_wait(k):
        pltpu.make_async_copy(a0_b.at[k], alpha0_hbm.at[pl.ds(0, CH)],
                              sem_o.at[k]).wait()
        pltpu.make_async_copy(a1_b.at[k], alpha1_hbm.at[pl.ds(0, CH)],
                              sem_o.at[k]).wait()

    idx_start(0, 0)
    idx_start(1, 1)
    idx_wait(0)
    gather_start(0)

    def outer_body(b, _):
        for k in range(RING):
            c = RING * b + k
            k1 = (k + 1) % RING
            k2 = (k + 2) % RING

            @pl.when(c + 2 < NCH)
            def _():
                idx_start(c + 2, k2)

            @pl.when(c + 1 < NCH)
            def _():
                idx_wait(k1)
                gather_start(k1)

            gather_wait(k)

            @pl.when(c >= RING)
            def _():
                out_wait(k)

            def group_body(g, _):
                row0 = CH * k + L * g
                ea = [ea_b[k, q, pl.ds(L * g, 16)] for q in range(4)]
                for q in range(4):
                    easum_v[pl.ds(16 * q, 16)] = \
                        easum_v[pl.ds(16 * q, 16)] + ea[q]
                acc0 = jnp.zeros((L,), jnp.float32)
                acc1 = jnp.zeros((L,), jnp.float32)
                for j in range(L):
                    t = [jnp.take(ea[q], ctake[j]) for q in range(4)]
                    a0 = jnp.float32(0)
                    a1 = jnp.float32(0)
                    for v in range(4):
                        ef = (t[0] * we[0][v] + t[1] * we[1][v]
                              + t[2] * we[2][v] + t[3] * we[3][v])
                        m = xl_rows[row0 + j, pl.ds(16 * v, 16)] \
                            + xr_rows[row0 + j, pl.ds(16 * v, 16)] + ef
                        lr = jnp.maximum(m, 0.2 * m)
                        if v < 2:
                            a0 = a0 + jnp.sum(lr * at[0][v], axis=0)
                        else:
                            a1 = a1 + jnp.sum(lr * at[1][v - 2], axis=0)
                    acc0 = jnp.where(lane == j, a0, acc0)
                    acc1 = jnp.where(lane == j, a1, acc1)
                a0_b[k, pl.ds(L * g, L)] = acc0
                a1_b[k, pl.ds(L * g, L)] = acc1

                dvec = ei_b[k, 1, pl.ds(L * g, 16)]
                for h, acc in ((0, acc0), (1, acc1)):
                    kk, vv = plsc.sort_key_val(2 * dvec + h, acc)
                    for sh in (1, 2, 4, 8):
                        sl = jnp.maximum(lane - sh, 0)
                        same = (jnp.take(kk, sl) == kk) & (lane >= sh)
                        vv = jnp.where(same,
                                       jnp.maximum(vv, jnp.take(vv, sl)), vv)
                    nxt = jnp.minimum(lane + 1, L - 1)
                    last = (jnp.take(kk, nxt) != kk) | (lane == L - 1)
                    cur = plsc.load_gather(amax_priv, [kk])
                    plsc.store_scatter(amax_priv, [kk],
                                       jnp.maximum(cur, vv), mask=last)
                return 0

            lax.fori_loop(0, CG, group_body, 0)
            out_start(c, k)
        return 0

    lax.fori_loop(0, NCH // RING, outer_body, 0)
    for k in range(RING):
        out_wait(k)
    pltpu.sync_copy(easum_v, easum_hbm.at[pl.ds(64 * wid, 64)])

    pltpu.sync_copy(amax_priv, amax_sh.at[sid])
    plsc.subcore_barrier()

    def comb_start(i, s):
        pltpu.async_copy(amax_sh.at[:, pl.ds(CHK * sid + 64 * i, 64)],
                         xl_rows.at[pl.ds(16 * s, 16)], sem_g.at[s])

    def comb_wait(s):
        pltpu.make_async_copy(amax_sh.at[:, pl.ds(0, 64)],
                              xl_rows.at[pl.ds(16 * s, 16)],
                              sem_g.at[s]).wait()

    comb_start(0, 0)

    def comb_outer(tt, _):
        for s in range(2):
            i = 2 * tt + s
            comb_wait(s)

            @pl.when(i + 1 < CHK // 64)
            def _():
                comb_start(i + 1, 1 - s)

            for v in range(4):
                m = xl_rows[16 * s, pl.ds(16 * v, 16)]
                for r in range(1, NS):
                    m = jnp.maximum(m, xl_rows[16 * s + r, pl.ds(16 * v, 16)])
                amax_priv[pl.ds(64 * i + 16 * v, 16)] = m
        return 0

    lax.fori_loop(0, CHK // 128, comb_outer, 0)
    pltpu.sync_copy(amax_priv.at[pl.ds(0, CHK)],
                    amax_hbm.at[pl.ds(cid * AMX + CHK * sid, CHK)])


def _sc_alpha(ei, ea_T, xl, xr, W_e, att):
    return pl.kernel(
        _sc_alpha_body,
        name="sc_alpha",
        out_type=[
            jax.ShapeDtypeStruct((E,), jnp.float32),
            jax.ShapeDtypeStruct((E,), jnp.float32),
            jax.ShapeDtypeStruct((NC * AMX,), jnp.float32),
            jax.ShapeDtypeStruct((NW * 64,), jnp.float32),
        ],
        mesh=_mesh,
        scratch_types=[
            pltpu.VMEM((RING, 2, CH), jnp.int32),
            pltpu.VMEM((RING, 4, CH), jnp.float32),
            pltpu.VMEM((RING * CH, D), jnp.float32),
            pltpu.VMEM((RING * CH, D), jnp.float32),
            pltpu.VMEM((RING, CH), jnp.float32),
            pltpu.VMEM((RING, CH), jnp.float32),
            pltpu.VMEM((AMX,), jnp.float32),
            pltpu.VMEM((64,), jnp.float32),
            pltpu.VMEM((4, D), jnp.float32),
            pltpu.VMEM((2, 32), jnp.float32),
            pltpu.VMEM_SHARED((NS, AMX), jnp.float32),
            pltpu.SemaphoreType.DMA((RING,)),
            pltpu.SemaphoreType.DMA((RING,)),
            pltpu.SemaphoreType.DMA((RING,)),
        ],
        compiler_params=_sc_params,
    )(ei, ea_T, xl, xr, W_e, att)


def _sc_scatter_body(ei_hbm, alpha0_hbm, alpha1_hbm, amax_hbm, xl_hbm,
                     numer_hbm,
                     ei_b, xl_rows, msg_rows, a0_b, a1_b, amax_loc,
                     numer_sp, sem_i, sem_g, sem_sc):
    cid = lax.axis_index("c")
    sid = lax.axis_index("s")
    wid = sid * NC + cid
    lane = lax.iota(jnp.int32, L)
    cj = [jnp.full((L,), j, jnp.int32) for j in range(L)]

    def zero_body(r, _):
        for v in range(5):
            msg_rows[r, pl.ds(16 * v, 16)] = jnp.zeros((L,), jnp.float32)
        return 0

    lax.fori_loop(0, RING * CH, zero_body, 0)
    npt = N // NS
    pltpu.sync_copy(msg_rows.at[pl.ds(0, 400)],
                    numer_sp.at[pl.ds(npt * sid, 400)])
    pltpu.sync_copy(msg_rows.at[pl.ds(0, npt - 400)],
                    numer_sp.at[pl.ds(npt * sid + 400, npt - 400)])

    pltpu.sync_copy(amax_hbm.at[pl.ds(cid * AMX, AMX)], amax_loc)
    plsc.subcore_barrier()

    def idx_start(c, k):
        base = wid * EPT + CH * c
        pltpu.async_copy(ei_hbm.at[:, pl.ds(base, CH)], ei_b.at[k],
                         sem_i.at[k])
        pltpu.async_copy(alpha0_hbm.at[pl.ds(base, CH)], a0_b.at[k],
                         sem_i.at[k])
        pltpu.async_copy(alpha1_hbm.at[pl.ds(base, CH)], a1_b.at[k],
                         sem_i.at[k])

    def idx_wait(k):
        pltpu.make_async_copy(ei_hbm.at[:, pl.ds(0, CH)], ei_b.at[k],
                              sem_i.at[k]).wait()
        pltpu.make_async_copy(alpha0_hbm.at[pl.ds(0, CH)], a0_b.at[k],
                              sem_i.at[k]).wait()
        pltpu.make_async_copy(alpha1_hbm.at[pl.ds(0, CH)], a1_b.at[k],
                              sem_i.at[k]).wait()

    def gather_start(k):
        pltpu.async_copy(xl_hbm.at[ei_b.at[k, 0]],
                         xl_rows.at[pl.ds(CH * k, CH)], sem_g.at[k])

    def gather_wait(k):
        pltpu.make_async_copy(xl_hbm.at[ei_b.at[k, 0]],
                              xl_rows.at[pl.ds(CH * k, CH)],
                              sem_g.at[k]).wait()

    def scatter_start(k):
        pltpu.async_copy(msg_rows.at[pl.ds(CH * k, CH)],
                         numer_sp.at[ei_b.at[k, 1]], sem_sc.at[k], add=True)

    def scatter_wait(k):
        pltpu.make_async_copy(msg_rows.at[pl.ds(CH * k, CH)],
                              numer_sp.at[ei_b.at[k, 1]],
                              sem_sc.at[k]).wait()

    idx_start(0, 0)
    idx_start(1, 1)
    idx_wait(0)
    gather_start(0)

    def outer_body(b, _):
        for k in range(RING):
            c = RING * b + k
            k1 = (k + 1) % RING
            k2 = (k + 2) % RING

            @pl.when(c + 2 < NCH)
            def _():
                @pl.when(c >= 3)
                def _():
                    scatter_wait(k2)

                idx_start(c + 2, k2)

            @pl.when(c + 1 < NCH)
            def _():
                idx_wait(k1)
                gather_start(k1)

            gather_wait(k)

            def group_body(g, _):
                row0 = CH * k + L * g
                dvec = ei_b[k, 1, pl.ds(L * g, 16)]
                am0 = plsc.load_gather(amax_loc, [2 * dvec])
                am1 = plsc.load_gather(amax_loc, [2 * dvec + 1])
                w0 = jnp.exp(a0_b[k, pl.ds(L * g, L)] - am0)
                w1 = jnp.exp(a1_b[k, pl.ds(L * g, L)] - am1)
                for j in range(L):
                    e = row0 + j
                    w0j = jnp.take(w0, cj[j])
                    w1j = jnp.take(w1, cj[j])
                    msg_rows[e, pl.ds(0, 16)] = xl_rows[e, pl.ds(0, 16)] * w0j
                    msg_rows[e, pl.ds(16, 16)] = \
                        xl_rows[e, pl.ds(16, 16)] * w0j
                    msg_rows[e, pl.ds(32, 16)] = \
                        xl_rows[e, pl.ds(32, 16)] * w1j
                    msg_rows[e, pl.ds(48, 16)] = \
                        xl_rows[e, pl.ds(48, 16)] * w1j
                    den = jnp.where(lane == 0, w0j,
                                    jnp.where(lane == 1, w1j, 0.0))
                    msg_rows[e, pl.ds(64, 16)] = den
                return 0

            lax.fori_loop(0, CG, group_body, 0)
            scatter_start(k)
        return 0

    lax.fori_loop(0, NCH // RING, outer_body, 0)
    for k in range(RING):
        scatter_wait(k)
    plsc.subcore_barrier()

    @pl.when(sid == 0)
    def _():
        pltpu.sync_copy(numer_sp, numer_hbm.at[pl.ds(cid * N, N)])


def _sc_scatter(ei, alpha0, alpha1, amax_sc, xl):
    return pl.kernel(
        _sc_scatter_body,
        name="sc_scatter",
        out_type=jax.ShapeDtypeStruct((NC * N, 80), jnp.float32),
        mesh=_mesh,
        scratch_types=[
            pltpu.VMEM((RING, 2, CH), jnp.int32),
            pltpu.VMEM((RING * CH, D), jnp.float32),
            pltpu.VMEM((RING * CH, 80), jnp.float32),
            pltpu.VMEM((RING, CH), jnp.float32),
            pltpu.VMEM((RING, CH), jnp.float32),
            pltpu.VMEM((AMX,), jnp.float32),
            pltpu.VMEM_SHARED((N, 80), jnp.float32),
            pltpu.SemaphoreType.DMA((RING,)),
            pltpu.SemaphoreType.DMA((RING,)),
            pltpu.SemaphoreType.DMA((RING,)),
        ],
        compiler_params=_sc_params,
    )(ei, alpha0, alpha1, amax_sc, xl)


def _final_body(numer_ref, xl_ref, xr_ref, amaxsc_ref, easum_ref, fsel_ref,
                att_ref, bias_ref, wout_ref, bout_ref, z_ref):
    num0 = numer_ref[0]
    num1 = numer_ref[1]
    pm0 = amaxsc_ref[0]
    pm1 = amaxsc_ref[1]

    s1 = jnp.sum(easum_ref[...], axis=0, keepdims=True)
    efm = jnp.dot(s1, fsel_ref[...]) * (1.0 / E)
    m = xl_ref[...] + xr_ref[...] + efm
    lr = jnp.where(m >= 0, m, 0.2 * m)
    a0 = jnp.sum(lr[:, 0:32] * att_ref[0:1, :], axis=1, keepdims=True)
    a1 = jnp.sum(lr[:, 32:64] * att_ref[1:2, :], axis=1, keepdims=True)
    aself = jnp.concatenate([a0, a1], axis=1)

    amax = jnp.maximum(jnp.maximum(pm0, pm1), aself)
    s0 = jnp.exp(pm0 - amax)
    s1 = jnp.exp(pm1 - amax)
    w = jnp.exp(aself - amax)
    w0 = w[:, 0:1]
    w1 = w[:, 1:2]
    xlv = xl_ref[...]
    n0 = (num0[:, 0:32] * s0[:, 0:1] + num1[:, 0:32] * s1[:, 0:1]
          + w0 * xlv[:, 0:32])
    n1 = (num0[:, 32:64] * s0[:, 1:2] + num1[:, 32:64] * s1[:, 1:2]
          + w1 * xlv[:, 32:64])
    d0 = num0[:, 64:65] * s0[:, 0:1] + num1[:, 64:65] * s1[:, 0:1] + w0 + 1e-16
    d1 = num0[:, 65:66] * s0[:, 1:2] + num1[:, 65:66] * s1[:, 1:2] + w1 + 1e-16
    g = jnp.concatenate([n0 / d0, n1 / d1], axis=1) + bias_ref[...]
    hh = jnp.where(g > 0, g, jnp.exp(g) - 1.0)
    z_ref[...] = jnp.dot(hh, wout_ref[...]) + bout_ref[...]


BN = 1000


def _final(numer3, xl, xr, amax_sc, easum, Fsel, att, bias_conv, W_out,
           b_out):
    return pl.pallas_call(
        _final_body,
        grid=(N // BN,),
        in_specs=[
            pl.BlockSpec((NC, BN, 80), lambda i: (0, i, 0)),
            pl.BlockSpec((BN, D), lambda i: (i, 0)),
            pl.BlockSpec((BN, D), lambda i: (i, 0)),
            pl.BlockSpec((NC, BN, 2), lambda i: (0, i, 0)),
            pl.BlockSpec((NW, 64), lambda i: (0, 0)),
            pl.BlockSpec((64, D), lambda i: (0, 0)),
            pl.BlockSpec((2, 32), lambda i: (0, 0)),
            pl.BlockSpec((1, D), lambda i: (0, 0)),
            pl.BlockSpec((D, OUT_DIM), lambda i: (0, 0)),
            pl.BlockSpec((1, OUT_DIM), lambda i: (0, 0)),
        ],
        out_specs=pl.BlockSpec((BN, OUT_DIM), lambda i: (i, 0)),
        out_shape=jax.ShapeDtypeStruct((N, OUT_DIM), jnp.float32),
        compiler_params=pltpu.CompilerParams(vmem_limit_bytes=100 * 1024 * 1024),
    )(numer3, xl, xr, amax_sc, easum, Fsel, att, bias_conv, W_out, b_out)


def kernel(x, edge_index, edge_attr, W_in, b_in, W_l, b_l, W_r, b_r, W_e,
           att, bias_conv, W_out, b_out):
    ea_T = edge_attr.T
    Fsel = jnp.repeat(W_e, 16, axis=0)

    xl, xr = _proj(x, W_in, b_in.reshape(1, D), W_l, b_l.reshape(1, D),
                   W_r, b_r.reshape(1, D))
    alpha0, alpha1, amax_sc, easum = _sc_alpha(edge_index, ea_T, xl, xr,
                                               W_e, att)
    numer_part = _sc_scatter(edge_index, alpha0, alpha1, amax_sc, xl)
    z = _final(numer_part.reshape(NC, N, 80), xl, xr,
               amax_sc.reshape(NC, NPAD, 2), easum.reshape(NW, 64), Fsel,
               att, bias_conv.reshape(1, D), W_out,
               b_out.reshape(1, OUT_DIM))
    return z
